# baseline (speedup 1.0000x reference)
import functools

import jax
import jax.numpy as jnp
from jax import lax
from jax.experimental import pallas as pl
from jax.experimental.pallas import tpu as pltpu
from jax.experimental.pallas import tpu_sc as plsc

_N1 = 85
_N1P = 96
_E1 = 2720
_E1P = 2816
_N2 = 5625
_N2P = 5632
_E2 = 180000
_E2P = 180224
_EPT = _E2P // 16
_NVE = _EPT // 16
_SLC = _N2P // 16
_NVS = _SLC // 16
_NVN = _N2P // 16


def _tc_dense(ei_ref, eit_ref, x1_ref, w1_ref, b1_ref, x2_ref, w2_ref,
              h1_ref, h2_ref):
    src_row = ei_ref[pl.ds(0, 1), :]
    dst_row = ei_ref[pl.ds(1, 1), :]
    src_col = eit_ref[:, pl.ds(0, 1)]
    node_r = lax.broadcasted_iota(jnp.int32, (_N1P, _E1P), 0)
    node_c = lax.broadcasted_iota(jnp.int32, (_E1P, _N1P), 1)
    od_t = jnp.where(node_r == dst_row, 1.0, 0.0)
    os_ = jnp.where(node_c == src_col, 1.0, 0.0)
    acore = jnp.dot(od_t, os_, preferred_element_type=jnp.float32)
    deg = jnp.sum(od_t, axis=1, keepdims=True)
    dinv = lax.rsqrt(deg + 1.0)
    r0 = lax.broadcasted_iota(jnp.int32, (_N1P, _N1P), 0)
    r1 = lax.broadcasted_iota(jnp.int32, (_N1P, _N1P), 1)
    eye = jnp.where((r0 == r1) & (r0 < _N1), 1.0, 0.0)
    c = jnp.dot(x1_ref[...], w1_ref[...], preferred_element_type=jnp.float32)
    m = jnp.dot(acore + eye, dinv * c, preferred_element_type=jnp.float32)
    h1_ref[...] = dinv * m + b1_ref[...]
    h2_ref[...] = jnp.dot(x2_ref[...], w2_ref[...],
                          preferred_element_type=jnp.float32)
    _ = src_row


def _tc_final(xc_ref, wf_ref, bf_ref, o_ref):
    x = jnp.maximum(xc_ref[...], 0.0)
    o_ref[...] = (jnp.dot(x, wf_ref[...], preferred_element_type=jnp.float32)
                  + bf_ref[...])


def _rsqrt16(x):
    i = plsc.bitcast(x, jnp.int32)
    i = jnp.int32(0x5F3759DF) - lax.shift_right_logical(i, 1)
    y = plsc.bitcast(i, jnp.float32)
    for _ in range(3):
        y = y * (1.5 - 0.5 * x * y * y)
    return y


_RPT = _EPT // 128


_CHK = 4
_CSZ = _EPT // _CHK


_OFF_B2 = 2 * _N2P
_OFF_ONES = 2 * _N2P + 32
_OFF_WFB = 2 * _N2P + 32 + _EPT
_FIN = 5 * _SLC
_OPT = 2048 // 16


def _sc_agg_body(ei_hbm, fdata_hbm, out_hbm,
                 src_v, dst_v, g_v, red_v, dinv_v, tsl_v,
                 hsl_v, b2_v, out_v, wfb_v, zb_v, val_v, idx_v, wsl_v,
                 sem_a, sem_b,
                 s_deg, s_t, s_out):
    c = lax.axis_index("c")
    s = lax.axis_index("s")
    nbase = s * _SLC
    ebase = s * _EPT
    st0 = pltpu.async_copy(ei_hbm.at[pl.ds(_E2P + ebase, _EPT)], dst_v, sem_a)
    st1 = pltpu.async_copy(fdata_hbm.at[pl.ds(_OFF_ONES, _EPT)], g_v, sem_b)
    pltpu.sync_copy(fdata_hbm.at[pl.ds(_OFF_B2 + c * 16, 16)], b2_v)
    pltpu.sync_copy(fdata_hbm.at[pl.ds(_OFF_WFB, 360)], wfb_v)

    zero16 = jnp.zeros((16,), jnp.float32)

    def _zero_body(i, carry):
        out_v[pl.ds(i * 16, 16)] = zero16
        return carry

    def _zero_body2(i, carry):
        zb_v[pl.ds(i * 16, 16)] = zero16
        return carry

    lax.fori_loop(0, _NVS, _zero_body, 0)
    lax.fori_loop(0, _OPT // 16, _zero_body2, 0)
    pltpu.sync_copy(out_v, s_deg.at[pl.ds(nbase, _SLC)])
    pltpu.sync_copy(zb_v, s_out.at[pl.ds(s * _OPT, _OPT)])

    i16 = lax.broadcasted_iota(jnp.int32, (16,), 0)

    def _idx_body(k, carry):
        f = 2 * (i16 + (nbase + k * 16)) + c
        i_ = lax.shift_right_logical(f * 46604, 21)
        wbase = (f - 45 * i_) * 8
        pos = i_ * 8
        for j in range(5):
            sl = pl.ds(j * _SLC + k * 16, 16)
            wsl_v[sl] = plsc.load_gather(wfb_v, [wbase + j])
            idx_v[sl] = pos + j
        return carry

    lax.fori_loop(0, _NVS, _idx_body, 0)
    st0.wait()
    st1.wait()
    plsc.subcore_barrier()

    st1 = pltpu.async_copy(ei_hbm.at[pl.ds(ebase, _EPT)], src_v, sem_a)
    st2 = pltpu.async_copy(fdata_hbm.at[pl.ds(c * _N2P + nbase, _SLC)],
                           hsl_v, sem_b)
    pltpu.sync_copy(g_v, s_deg.at[dst_v], add=True)
    st1.wait()
    st2.wait()
    plsc.subcore_barrier()

    pltpu.sync_copy(s_deg.at[pl.ds(nbase, _SLC)], red_v)

    def _dinv_body(k, carry):
        sl = pl.ds(k * 16, 16)
        y = _rsqrt16(red_v[sl] + 1.0)
        dinv_v[sl] = y
        tsl_v[sl] = y * hsl_v[sl]
        return carry

    lax.fori_loop(0, _NVS, _dinv_body, 0)
    pltpu.sync_copy(tsl_v, s_t.at[pl.ds(nbase, _SLC)])
    plsc.subcore_barrier()

    pltpu.sync_copy(s_t.at[src_v], g_v)
    pltpu.sync_copy(g_v, s_deg.at[dst_v], add=True)
    plsc.subcore_barrier()

    pltpu.sync_copy(s_deg.at[pl.ds(nbase, _SLC)], hsl_v)
    b2c = b2_v[...]

    def _fin_body(k, carry):
        sl = pl.ds(k * 16, 16)
        y = dinv_v[sl]
        msg = hsl_v[sl] - red_v[sl]
        v = jnp.maximum(y * msg + y * tsl_v[sl] + b2c, 0.0)
        for j in range(5):
            vsl = pl.ds(j * _SLC + k * 16, 16)
            val_v[vsl] = v * wsl_v[vsl]
        return carry

    lax.fori_loop(0, _NVS, _fin_body, 0)
    pltpu.sync_copy(val_v, s_out.at[idx_v], add=True)
    plsc.subcore_barrier()
    pltpu.sync_copy(s_out.at[pl.ds(s * _OPT, _OPT)], zb_v)
    pltpu.sync_copy(zb_v, out_hbm.at[pl.ds(c * 2048 + s * _OPT, _OPT)])


_sc_agg = functools.partial(
    pl.kernel,
    mesh=plsc.VectorSubcoreMesh(core_axis_name="c", subcore_axis_name="s"),
    out_type=jax.ShapeDtypeStruct((2 * 2048,), jnp.float32),
    compiler_params=pltpu.CompilerParams(needs_layout_passes=False),
    scratch_types=[
        pltpu.VMEM((_EPT,), jnp.int32),
        pltpu.VMEM((_EPT,), jnp.int32),
        pltpu.VMEM((_EPT,), jnp.float32),
        pltpu.VMEM((_SLC,), jnp.float32),
        pltpu.VMEM((_SLC,), jnp.float32),
        pltpu.VMEM((_SLC,), jnp.float32),
        pltpu.VMEM((_SLC,), jnp.float32),
        pltpu.VMEM((16,), jnp.float32),
        pltpu.VMEM((_SLC,), jnp.float32),
        pltpu.VMEM((360,), jnp.float32),
        pltpu.VMEM((_OPT,), jnp.float32),
        pltpu.VMEM((_FIN,), jnp.float32),
        pltpu.VMEM((_FIN,), jnp.int32),
        pltpu.VMEM((_FIN,), jnp.float32),
        pltpu.SemaphoreType.DMA,
        pltpu.SemaphoreType.DMA,
        pltpu.VMEM_SHARED((_N2P,), jnp.float32),
        pltpu.VMEM_SHARED((_N2P,), jnp.float32),
        pltpu.VMEM_SHARED((2048,), jnp.float32),
    ],
)(_sc_agg_body)


def kernel(x1, edge_index1, x2, edge_index2, W1, b1, W2, b2, Wf, bf):
    f32 = jnp.float32
    ei1 = edge_index1.astype(jnp.int32)
    pad1 = jnp.full((2, _E1P - _E1), _N1P - 1, jnp.int32)
    ei1p = jnp.concatenate([ei1, pad1], axis=1)
    x1p = jnp.pad(x1, ((0, _N1P - _N1), (0, 0)))
    w1p = jnp.pad(W1, ((0, 0), (0, 256 - 200)))
    b1p = jnp.pad(b1, (0, 256 - 200)).reshape(1, 256)
    x2p = jnp.pad(x2, ((0, _N2P - _N2), (0, 0)))
    w2p = jnp.pad(W2, ((0, 0), (0, 8 - 2)))

    h1, h2 = pl.pallas_call(
        _tc_dense,
        out_shape=[
            jax.ShapeDtypeStruct((_N1P, 256), f32),
            jax.ShapeDtypeStruct((_N2P, 8), f32),
        ],
    )(ei1p, ei1p.T, x1p, w1p, b1p, x2p, w2p)

    ei2 = edge_index2.astype(jnp.int32)
    pad2 = jnp.full((2, _E2P - _E2), _N2P - 1, jnp.int32)
    ei2p = jnp.concatenate([ei2, pad2], axis=1)
    h_t = h2[:, :2].T.reshape(2 * _N2P)
    b2b = jnp.broadcast_to(b2.reshape(2, 1), (2, 16)).reshape(32)
    ones1 = jnp.ones((_EPT,), jnp.float32)
    wfb = jnp.pad(Wf[68:113, :], ((0, 0), (0, 3))).reshape(360)
    ei_flat = ei2p.reshape(2 * _E2P)
    fdata = jnp.concatenate([h_t, b2b, ones1, wfb])
    sout = _sc_agg(ei_flat, fdata)

    r1 = h1[:_N1, :200].reshape(250, 68)
    r1p = jnp.pad(r1, ((0, 6), (0, 128 - 68)))
    wtp = jnp.pad(Wf[:68, :], ((0, 128 - 68), (0, 3)))
    bfp = jnp.pad(bf, (0, 3)).reshape(1, 8)
    p1 = pl.pallas_call(
        _tc_final,
        out_shape=jax.ShapeDtypeStruct((256, 8), f32),
    )(r1p, wtp, bfp)
    o = p1 + sout[:2048].reshape(256, 8) + sout[2048:].reshape(256, 8)
    return o[:250, :5]


# speedup vs baseline: 1.0068x; 1.0068x over previous
import functools

import jax
import jax.numpy as jnp
from jax import lax
from jax.experimental import pallas as pl
from jax.experimental.pallas import tpu as pltpu
from jax.experimental.pallas import tpu_sc as plsc

_N1 = 85
_N1P = 96
_E1 = 2720
_E1P = 2816
_N2 = 5625
_N2P = 5632
_E2 = 180000
_E2P = 180224
_EPT = _E2P // 16
_NVE = _EPT // 16
_SLC = _N2P // 16
_NVS = _SLC // 16
_NVN = _N2P // 16


def _tc_dense(ei_ref, eit_ref, x1_ref, w1_ref, b1_ref, x2_ref, w2_ref,
              h1_ref, h2_ref):
    src_row = ei_ref[pl.ds(0, 1), :]
    dst_row = ei_ref[pl.ds(1, 1), :]
    src_col = eit_ref[:, pl.ds(0, 1)]
    node_r = lax.broadcasted_iota(jnp.int32, (_N1P, _E1P), 0)
    node_c = lax.broadcasted_iota(jnp.int32, (_E1P, _N1P), 1)
    od_t = jnp.where(node_r == dst_row, 1.0, 0.0)
    os_ = jnp.where(node_c == src_col, 1.0, 0.0)
    acore = jnp.dot(od_t, os_, preferred_element_type=jnp.float32)
    deg = jnp.sum(od_t, axis=1, keepdims=True)
    dinv = lax.rsqrt(deg + 1.0)
    r0 = lax.broadcasted_iota(jnp.int32, (_N1P, _N1P), 0)
    r1 = lax.broadcasted_iota(jnp.int32, (_N1P, _N1P), 1)
    eye = jnp.where((r0 == r1) & (r0 < _N1), 1.0, 0.0)
    c = jnp.dot(x1_ref[...], w1_ref[...], preferred_element_type=jnp.float32)
    m = jnp.dot(acore + eye, dinv * c, preferred_element_type=jnp.float32)
    h1_ref[...] = dinv * m + b1_ref[...]
    h2_ref[...] = jnp.dot(x2_ref[...], w2_ref[...],
                          preferred_element_type=jnp.float32)
    _ = src_row


def _tc_final(xc_ref, wf_ref, bf_ref, o_ref):
    x = jnp.maximum(xc_ref[...], 0.0)
    o_ref[...] = (jnp.dot(x, wf_ref[...], preferred_element_type=jnp.float32)
                  + bf_ref[...])


def _rsqrt16(x):
    i = plsc.bitcast(x, jnp.int32)
    i = jnp.int32(0x5F3759DF) - lax.shift_right_logical(i, 1)
    y = plsc.bitcast(i, jnp.float32)
    for _ in range(3):
        y = y * (1.5 - 0.5 * x * y * y)
    return y


_RPT = _EPT // 128


_CHK = 4
_CSZ = _EPT // _CHK


_OFF_B2 = 2 * _N2P
_OFF_ONES = 2 * _N2P + 32
_OFF_WFB = 2 * _N2P + 32 + _EPT
_FIN = 5 * _SLC
_OPT = 2048 // 16


def _sc_agg_body(ei_hbm, fdata_hbm, out_hbm,
                 src_v, dst_v, g_v, red_v, dinv_v, tsl_v,
                 hsl_v, b2_v, out_v, wfb_v, zb_v, val_v, idx_v,
                 sem_a, sem_b,
                 s_deg, s_t, s_out):
    c = lax.axis_index("c")
    s = lax.axis_index("s")
    nbase = s * _SLC
    ebase = s * _EPT
    st0 = pltpu.async_copy(ei_hbm.at[pl.ds(_E2P + ebase, _EPT)], dst_v, sem_a)
    st1 = pltpu.async_copy(fdata_hbm.at[pl.ds(_OFF_ONES, _EPT)], g_v, sem_b)
    pltpu.sync_copy(fdata_hbm.at[pl.ds(_OFF_B2 + c * 16, 16)], b2_v)
    pltpu.sync_copy(fdata_hbm.at[pl.ds(_OFF_WFB, 360)], wfb_v)

    zero16 = jnp.zeros((16,), jnp.float32)

    def _zero_body(i, carry):
        out_v[pl.ds(i * 16, 16)] = zero16
        return carry

    def _zero_body2(i, carry):
        zb_v[pl.ds(i * 16, 16)] = zero16
        return carry

    lax.fori_loop(0, _NVS, _zero_body, 0)
    lax.fori_loop(0, _OPT // 16, _zero_body2, 0)
    pltpu.sync_copy(out_v, s_deg.at[pl.ds(nbase, _SLC)])
    pltpu.sync_copy(zb_v, s_out.at[pl.ds(s * _OPT, _OPT)])

    st0.wait()
    st1.wait()
    plsc.subcore_barrier()

    st1 = pltpu.async_copy(ei_hbm.at[pl.ds(ebase, _EPT)], src_v, sem_a)
    st2 = pltpu.async_copy(fdata_hbm.at[pl.ds(c * _N2P + nbase, _SLC)],
                           hsl_v, sem_b)
    pltpu.sync_copy(g_v, s_deg.at[dst_v], add=True)
    st1.wait()
    st2.wait()
    plsc.subcore_barrier()

    pltpu.sync_copy(s_deg.at[pl.ds(nbase, _SLC)], red_v)

    def _dinv_body(k, carry):
        sl = pl.ds(k * 16, 16)
        y = _rsqrt16(red_v[sl] + 1.0)
        dinv_v[sl] = y
        tsl_v[sl] = y * hsl_v[sl]
        return carry

    lax.fori_loop(0, _NVS, _dinv_body, 0)
    pltpu.sync_copy(tsl_v, s_t.at[pl.ds(nbase, _SLC)])
    plsc.subcore_barrier()

    pltpu.sync_copy(s_t.at[src_v], g_v)
    pltpu.sync_copy(g_v, s_deg.at[dst_v], add=True)
    plsc.subcore_barrier()

    pltpu.sync_copy(s_deg.at[pl.ds(nbase, _SLC)], hsl_v)
    b2c = b2_v[...]
    i16 = lax.broadcasted_iota(jnp.int32, (16,), 0)

    def _fin_body(k, carry):
        sl = pl.ds(k * 16, 16)
        y = dinv_v[sl]
        msg = hsl_v[sl] - red_v[sl]
        v = jnp.maximum(y * msg + y * tsl_v[sl] + b2c, 0.0)
        f = 2 * (i16 + (nbase + k * 16)) + c
        i_ = lax.shift_right_logical(f * 46604, 21)
        wbase = (f - 45 * i_) * 8
        pos = i_ * 8
        for j in range(5):
            vsl = pl.ds(j * _SLC + k * 16, 16)
            val_v[vsl] = v * plsc.load_gather(wfb_v, [wbase + j])
            idx_v[vsl] = pos + j
        return carry

    lax.fori_loop(0, _NVS, _fin_body, 0)
    pltpu.sync_copy(val_v, s_out.at[idx_v], add=True)
    plsc.subcore_barrier()
    pltpu.sync_copy(s_out.at[pl.ds(s * _OPT, _OPT)], zb_v)
    pltpu.sync_copy(zb_v, out_hbm.at[pl.ds(c * 2048 + s * _OPT, _OPT)])


_sc_agg = functools.partial(
    pl.kernel,
    mesh=plsc.VectorSubcoreMesh(core_axis_name="c", subcore_axis_name="s"),
    out_type=jax.ShapeDtypeStruct((2 * 2048,), jnp.float32),
    compiler_params=pltpu.CompilerParams(needs_layout_passes=False),
    scratch_types=[
        pltpu.VMEM((_EPT,), jnp.int32),
        pltpu.VMEM((_EPT,), jnp.int32),
        pltpu.VMEM((_EPT,), jnp.float32),
        pltpu.VMEM((_SLC,), jnp.float32),
        pltpu.VMEM((_SLC,), jnp.float32),
        pltpu.VMEM((_SLC,), jnp.float32),
        pltpu.VMEM((_SLC,), jnp.float32),
        pltpu.VMEM((16,), jnp.float32),
        pltpu.VMEM((_SLC,), jnp.float32),
        pltpu.VMEM((360,), jnp.float32),
        pltpu.VMEM((_OPT,), jnp.float32),
        pltpu.VMEM((_FIN,), jnp.float32),
        pltpu.VMEM((_FIN,), jnp.int32),
        pltpu.SemaphoreType.DMA,
        pltpu.SemaphoreType.DMA,
        pltpu.VMEM_SHARED((_N2P,), jnp.float32),
        pltpu.VMEM_SHARED((_N2P,), jnp.float32),
        pltpu.VMEM_SHARED((2048,), jnp.float32),
    ],
)(_sc_agg_body)


def kernel(x1, edge_index1, x2, edge_index2, W1, b1, W2, b2, Wf, bf):
    f32 = jnp.float32
    ei1 = edge_index1.astype(jnp.int32)
    pad1 = jnp.full((2, _E1P - _E1), _N1P - 1, jnp.int32)
    ei1p = jnp.concatenate([ei1, pad1], axis=1)
    x1p = jnp.pad(x1, ((0, _N1P - _N1), (0, 0)))
    w1p = jnp.pad(W1, ((0, 0), (0, 256 - 200)))
    b1p = jnp.pad(b1, (0, 256 - 200)).reshape(1, 256)
    x2p = jnp.pad(x2, ((0, _N2P - _N2), (0, 0)))
    w2p = jnp.pad(W2, ((0, 0), (0, 8 - 2)))

    h1, h2 = pl.pallas_call(
        _tc_dense,
        out_shape=[
            jax.ShapeDtypeStruct((_N1P, 256), f32),
            jax.ShapeDtypeStruct((_N2P, 8), f32),
        ],
    )(ei1p, ei1p.T, x1p, w1p, b1p, x2p, w2p)

    ei2 = edge_index2.astype(jnp.int32)
    pad2 = jnp.full((2, _E2P - _E2), _N2P - 1, jnp.int32)
    ei2p = jnp.concatenate([ei2, pad2], axis=1)
    h_t = h2[:, :2].T.reshape(2 * _N2P)
    b2b = jnp.broadcast_to(b2.reshape(2, 1), (2, 16)).reshape(32)
    ones1 = jnp.ones((_EPT,), jnp.float32)
    wfb = jnp.pad(Wf[68:113, :], ((0, 0), (0, 3))).reshape(360)
    ei_flat = ei2p.reshape(2 * _E2P)
    fdata = jnp.concatenate([h_t, b2b, ones1, wfb])
    sout = _sc_agg(ei_flat, fdata)

    r1 = h1[:_N1, :200].reshape(250, 68)
    r1p = jnp.pad(r1, ((0, 6), (0, 128 - 68)))
    wtp = jnp.pad(Wf[:68, :], ((0, 128 - 68), (0, 3)))
    bfp = jnp.pad(bf, (0, 3)).reshape(1, 8)
    p1 = pl.pallas_call(
        _tc_final,
        out_shape=jax.ShapeDtypeStruct((256, 8), f32),
    )(r1p, wtp, bfp)
    o = p1 + sout[:2048].reshape(256, 8) + sout[2048:].reshape(256, 8)
    return o[:250, :5]
